# baseline (speedup 1.0000x reference)
import functools

import jax
import jax.numpy as jnp
from jax import lax
from jax.experimental import pallas as pl
from jax.experimental.pallas import tpu as pltpu
from jax.experimental.pallas import tpu_sc as plsc

H = 256
F = 256
K = 3
NBINS = 256
SCALE = 10
B = 8
T = 2048
MAXLEN = 8192

_ZROW = B * T
_TBL_ROWS = (B + 1) * T


def _shifts(v):
    z = jnp.zeros((1, v.shape[1]), v.dtype)
    vm1 = jnp.concatenate([z, v[:-1]], axis=0)
    vp1 = jnp.concatenate([v[1:], z], axis=0)
    return vm1, vp1


def _ln(h, g, b):
    m = jnp.mean(h, axis=-1, keepdims=True)
    d = h - m
    v = jnp.mean(d * d, axis=-1, keepdims=True)
    return d * lax.rsqrt(v + 1e-5) * g + b


def _conv(x, xm1, xp1, w, b):
    y = jnp.dot(xm1, w[0], preferred_element_type=jnp.float32)
    y = y + jnp.dot(x, w[1], preferred_element_type=jnp.float32)
    y = y + jnp.dot(xp1, w[2], preferred_element_type=jnp.float32)
    return y + b


def _stack(x, xm1, xp1, c1, c2, vec, r0):
    h = jnp.maximum(_conv(x, xm1, xp1, c1, vec[r0:r0 + 1]), 0.0)
    h = _ln(h, vec[r0 + 1:r0 + 2], vec[r0 + 2:r0 + 3])
    hm1, hp1 = _shifts(h)
    h = jnp.maximum(_conv(h, hm1, hp1, c2, vec[r0 + 3:r0 + 4]), 0.0)
    h = _ln(h, vec[r0 + 4:r0 + 5], vec[r0 + 5:r0 + 6])
    return h


def _bin_onehot(tgt_col, bins_row):
    cnt = jnp.sum((bins_row < tgt_col).astype(jnp.float32), axis=-1,
                  keepdims=True)
    io = lax.broadcasted_iota(jnp.int32, (T, NBINS), 1).astype(jnp.float32)
    return (io == cnt).astype(jnp.float32)


def _to_col(v16):
    rs = (lax.broadcasted_iota(jnp.int32, (T, 16), 0) // 128
          == lax.broadcasted_iota(jnp.int32, (T, 16), 1)).astype(jnp.float32)
    pc = (lax.broadcasted_iota(jnp.int32, (T, 128), 0) % 128
          == lax.broadcasted_iota(jnp.int32, (T, 128), 1)).astype(jnp.float32)
    m1 = jnp.dot(rs, v16, preferred_element_type=jnp.float32)
    return jnp.sum(m1 * pc, axis=1, keepdims=True)


def _to_16(col):
    pc = (lax.broadcasted_iota(jnp.int32, (T, 128), 0) % 128
          == lax.broadcasted_iota(jnp.int32, (T, 128), 1)).astype(jnp.float32)
    rst = (lax.broadcasted_iota(jnp.int32, (16, T), 1) // 128
           == lax.broadcasted_iota(jnp.int32, (16, T), 0)).astype(jnp.float32)
    return jnp.dot(rst, col * pc, preferred_element_type=jnp.float32)


_WREFS = 7


def _tc_body(x_ref, mask_ref, pt_ref, et_ref, dur_ref, scal_ref,
             pbins_ref, ebins_ref, pemb_ref, eemb_ref, lw_ref, vec_ref,
             wd1_ref, wd2_ref, wp1_ref, wp2_ref, wps_ref, we1_ref, we2_ref,
             x3_ref, logd_ref, pitch_ref, energy_ref, gidx_ref, misc_ref,
             ws_ref):
    bid = pl.program_id(0)

    @pl.when(bid == 0)
    def _prologue():
        io_i = lax.broadcasted_iota(jnp.int32, (H, 3 * H), 0)
        io_r = lax.broadcasted_iota(jnp.int32, (H, 3 * H), 1)
        wrefs = (wd1_ref, wd2_ref, wp1_ref, wp2_ref, wps_ref, we1_ref,
                 we2_ref)
        for k in range(K):
            sel = (io_r == 3 * io_i + k).astype(jnp.float32)
            for i, wref in enumerate(wrefs):
                ws_ref[i, k] = lax.dot_general(
                    sel, wref[...], (((1,), (1,)), ((), ())),
                    preferred_element_type=jnp.float32)
        x3_ref[0] = jnp.zeros((T, H), jnp.float32)

    @pl.when(bid > 0)
    def _main():
        x = x_ref[0]
        inv_mask = 1.0 - _to_col(mask_ref[0])
        scal = scal_ref[...]
        lw = lw_ref[...]
        vec = vec_ref[...]
        xm1, xp1 = _shifts(x)

        hd = _stack(x, xm1, xp1, ws_ref[0], ws_ref[1], vec, 0)
        lind = jnp.dot(hd, lw, preferred_element_type=jnp.float32) + scal
        logd = lind[:, 0:1] * inv_mask

        hp = _stack(x, xm1, xp1, ws_ref[2], ws_ref[3], vec, 6)
        linp = jnp.dot(hp, lw, preferred_element_type=jnp.float32) + scal
        rec = linp[:, 2:3] * inv_mask
        sh = _conv(x, xm1, xp1, ws_ref[4], vec[18:19])
        sh_mean = jnp.mean(sh, axis=0, keepdims=True)
        stats = (jnp.dot(sh_mean, lw, preferred_element_type=jnp.float32)
                 + scal)
        pitch = rec + stats[:, 3:4] * stats[:, 4:5]

        ohp = _bin_onehot(_to_col(pt_ref[0]), pbins_ref[...])
        x2 = x + jnp.dot(ohp, pemb_ref[...],
                         preferred_element_type=jnp.float32)

        x2m1, x2p1 = _shifts(x2)
        he = _stack(x2, x2m1, x2p1, ws_ref[5], ws_ref[6], vec, 12)
        line = jnp.dot(he, lw, preferred_element_type=jnp.float32) + scal
        energy = line[:, 1:2] * inv_mask

        ohe = _bin_onehot(_to_col(et_ref[0]), ebins_ref[...])
        x3 = x2 + jnp.dot(ohe, eemb_ref[...],
                          preferred_element_type=jnp.float32)

        d16 = dur_ref[0]
        u128 = (lax.broadcasted_iota(jnp.int32, (128, 128), 0)
                <= lax.broadcasted_iota(jnp.int32, (128, 128), 1))
        rowc = jnp.dot(d16, u128.astype(jnp.float32),
                       preferred_element_type=jnp.float32)
        rtot = jnp.sum(d16, axis=1, keepdims=True)
        ls16 = (lax.broadcasted_iota(jnp.int32, (16, 16), 1)
                < lax.broadcasted_iota(jnp.int32, (16, 16), 0))
        excl16 = jnp.dot(ls16.astype(jnp.float32), rtot,
                         preferred_element_type=jnp.float32)
        mel = jnp.sum(rtot, axis=0, keepdims=True)
        cum = _to_col(rowc + excl16)
        cumv = cum - scal[:, 5:6]

        a = jnp.floor(cumv * (1.0 / 64.0))
        bfrac = cumv - 64.0 * a
        oh_hi = (lax.broadcasted_iota(jnp.int32, (T, 128), 1)
                 .astype(jnp.float32) == a)
        oh_lo = (lax.broadcasted_iota(jnp.int32, (T, 64), 1)
                 .astype(jnp.float32) == bfrac)
        hist2d = lax.dot_general(oh_hi.astype(jnp.float32),
                                 oh_lo.astype(jnp.float32),
                                 (((0,), (0,)), ((), ())),
                                 preferred_element_type=jnp.float32)
        u64 = (lax.broadcasted_iota(jnp.int32, (64, 64), 0)
               <= lax.broadcasted_iota(jnp.int32, (64, 64), 1))
        rowcum = jnp.dot(hist2d, u64.astype(jnp.float32),
                         preferred_element_type=jnp.float32)
        rtot2 = jnp.sum(hist2d, axis=1, keepdims=True)
        ls = (lax.broadcasted_iota(jnp.int32, (128, 128), 1)
              < lax.broadcasted_iota(jnp.int32, (128, 128), 0))
        excl = jnp.dot(ls.astype(jnp.float32), rtot2,
                       preferred_element_type=jnp.float32)
        negcnt = jnp.sum((cumv < 0.0).astype(jnp.float32), axis=0,
                         keepdims=True)
        idx2d = rowcum + excl + negcnt
        idxc = jnp.minimum(idx2d, float(T - 1)).astype(jnp.int32)

        j2d = (64 * lax.broadcasted_iota(jnp.int32, (128, 64), 0)
               + lax.broadcasted_iota(jnp.int32, (128, 64), 1)
               ).astype(jnp.float32)
        valid = j2d < (mel - scal[:, 5:6])
        zrow = _ZROW + lax.broadcasted_iota(jnp.int32, (128, 64), 1)
        gidx = jnp.where(valid, idxc + (bid - 1) * T, zrow)

        x3_ref[0] = x3
        logd_ref[0] = _to_16(logd)
        pitch_ref[0] = _to_16(pitch)
        energy_ref[0] = _to_16(energy)
        gidx_ref[0] = gidx
        misc_ref[0] = jnp.broadcast_to(mel, (1, 128))


def _tc_specs():
    bs = pl.BlockSpec
    bcast3 = lambda b: (0, 0, 0)
    bcast2 = lambda b: (0, 0)
    per_b3 = lambda b: (jnp.maximum(b - 1, 0), 0, 0)
    in_specs = [
        bs((1, T, H), per_b3),
        bs((1, 16, 128), per_b3),
        bs((1, 16, 128), per_b3),
        bs((1, 16, 128), per_b3),
        bs((1, 16, 128), per_b3),
        bs((1, 128), bcast2),
        bs((1, NBINS), bcast2),
        bs((1, NBINS), bcast2),
        bs((NBINS, H), bcast2),
        bs((NBINS, H), bcast2),
        bs((H, 128), bcast2),
        bs((24, H), bcast2),
    ] + [bs((H, 3 * H), bcast2)] * _WREFS
    out_specs = [
        bs((1, T, H), lambda b: (jnp.where(b == 0, B, b - 1), 0, 0)),
        bs((1, 16, 128), per_b3),
        bs((1, 16, 128), per_b3),
        bs((1, 16, 128), per_b3),
        bs((1, 128, 64), per_b3),
        bs((1, 1, 128), per_b3),
    ]
    out_shapes = [
        jax.ShapeDtypeStruct((B + 1, T, H), jnp.float32),
        jax.ShapeDtypeStruct((B, 16, 128), jnp.float32),
        jax.ShapeDtypeStruct((B, 16, 128), jnp.float32),
        jax.ShapeDtypeStruct((B, 16, 128), jnp.float32),
        jax.ShapeDtypeStruct((B, 128, 64), jnp.int32),
        jax.ShapeDtypeStruct((B, 1, 128), jnp.float32),
    ]
    return (B + 1,), in_specs, out_shapes, out_specs


def _prep(x, src_mask, pitch_target, energy_target, duration_target,
          max_len, params, pitch_bins, energy_bins):
    f32 = jnp.float32
    v16 = lambda v: v.astype(f32).reshape(B, 16, 128)
    isq = 1.0 / jnp.sqrt(jnp.arange(1, SCALE + 1, dtype=f32))

    pp, dp, ep = params['pitch'], params['dur'], params['energy']
    pvec = jnp.einsum('sf,s->f', pp['lin_w'][:SCALE], isq)
    pb = jnp.dot(pp['lin_b'][:SCALE], isq)
    lw = jnp.zeros((H, 128), f32)
    lw = lw.at[:, 0].set(dp['lin_w'][0])
    lw = lw.at[:, 1].set(ep['lin_w'][0])
    lw = lw.at[:, 2].set(pvec)
    lw = lw.at[:, 3].set(pp['slin_w'][0])
    lw = lw.at[:, 4].set(pp['slin_w'][1])

    offf = jnp.asarray(max_len, f32) - MAXLEN
    scal = jnp.zeros((128,), f32)
    scal = scal.at[0].set(dp['lin_b'][0])
    scal = scal.at[1].set(ep['lin_b'][0])
    scal = scal.at[2].set(pb)
    scal = scal.at[3].set(pp['slin_b'][0])
    scal = scal.at[4].set(pp['slin_b'][1])
    scal = scal.at[5].set(offf)
    scal = scal[None, :]

    rows = [dp['conv1_b'], dp['ln1_g'], dp['ln1_b'],
            dp['conv2_b'], dp['ln2_g'], dp['ln2_b'],
            pp['conv1_b'], pp['ln1_g'], pp['ln1_b'],
            pp['conv2_b'], pp['ln2_g'], pp['ln2_b'],
            ep['conv1_b'], ep['ln1_g'], ep['ln1_b'],
            ep['conv2_b'], ep['ln2_g'], ep['ln2_b'],
            pp['sconv_b']]
    vec = jnp.stack(rows + [jnp.zeros((H,), f32)] * (24 - len(rows)))

    rw = lambda w: w.astype(f32).reshape(H, 3 * H)
    pad_bins = lambda bb: jnp.concatenate(
        [bb.astype(f32), jnp.full((1,), jnp.inf, f32)])[None, :]

    return [
        x.astype(f32),
        v16(src_mask),
        v16(pitch_target),
        v16(energy_target),
        v16(duration_target),
        scal,
        pad_bins(pitch_bins),
        pad_bins(energy_bins),
        params['pitch_emb'].astype(f32),
        params['energy_emb'].astype(f32),
        lw,
        vec,
        rw(dp['conv1_w']), rw(dp['conv2_w']),
        rw(pp['conv1_w']), rw(pp['conv2_w']), rw(pp['sconv_w']),
        rw(ep['conv1_w']), rw(ep['conv2_w']),
    ]


_SC_NC = 2
_SC_NS = 16
_NW = _SC_NC * _SC_NS
_RPW = (B * MAXLEN) // _NW
_CH = 128
_NCHUNK = _RPW // _CH
_NBUF = 3


def _sc_gather(table, gidx):
    mesh = plsc.VectorSubcoreMesh(core_axis_name="c", subcore_axis_name="s")

    @functools.partial(
        pl.kernel, mesh=mesh,
        out_type=jax.ShapeDtypeStruct((B * MAXLEN, H), jnp.float32),
        scratch_types=[pltpu.VMEM((_NCHUNK, _CH), jnp.int32)]
        + [pltpu.VMEM((_CH, H), jnp.float32)] * _NBUF
        + [pltpu.SemaphoreType.DMA] * (2 * _NBUF),
    )
    def k(table_hbm, idx_hbm, out_hbm, idx_v, *bufs_and_sems):
        rows = bufs_and_sems[:_NBUF]
        gsem = bufs_and_sems[_NBUF:2 * _NBUF]
        ssem = bufs_and_sems[2 * _NBUF:]
        wid = lax.axis_index("s") * _SC_NC + lax.axis_index("c")
        base = wid * _RPW
        pltpu.sync_copy(idx_hbm.at[wid], idx_v)
        g_cp = [None] * _NCHUNK
        s_cp = [None] * _NCHUNK
        for c in range(_NCHUNK):
            b = c % _NBUF
            if c >= _NBUF:
                s_cp[c - _NBUF].wait()
            g_cp[c] = pltpu.async_copy(table_hbm.at[idx_v.at[c]], rows[b],
                                       gsem[b])
            if c >= 1:
                bp = (c - 1) % _NBUF
                g_cp[c - 1].wait()
                s_cp[c - 1] = pltpu.async_copy(
                    rows[bp], out_hbm.at[pl.ds(base + (c - 1) * _CH, _CH)],
                    ssem[bp])
        c = _NCHUNK - 1
        g_cp[c].wait()
        s_cp[c] = pltpu.async_copy(
            rows[c % _NBUF], out_hbm.at[pl.ds(base + c * _CH, _CH)],
            ssem[c % _NBUF])
        for c in range(_NCHUNK - _NBUF, _NCHUNK):
            s_cp[c].wait()

    return k(table, gidx)


def kernel(x, src_mask, pitch_target, energy_target, duration_target,
           max_len, params, pitch_bins, energy_bins):
    grid, in_specs, out_shapes, out_specs = _tc_specs()
    args = _prep(x, src_mask, pitch_target, energy_target, duration_target,
                 max_len, params, pitch_bins, energy_bins)
    x3, logd, pitch, energy, gidx, misc = pl.pallas_call(
        _tc_body,
        grid=grid,
        in_specs=in_specs,
        out_specs=out_specs,
        out_shape=out_shapes,
        scratch_shapes=[pltpu.VMEM((_WREFS, K, H, H), jnp.float32)],
        compiler_params=pltpu.CompilerParams(
            dimension_semantics=("arbitrary",)),
    )(*args)

    table = x3.reshape(_TBL_ROWS, H)
    gidx_3d = gidx.reshape(_NW, _NCHUNK, _CH)
    out = _sc_gather(table, gidx_3d).reshape(B, MAXLEN, H)

    mel_len = misc[:, 0, 0].astype(jnp.int32)
    return (out, pitch.reshape(B, T), energy.reshape(B, T),
            logd.reshape(B, T), mel_len)


# speedup vs baseline: 1.0892x; 1.0892x over previous
import functools

import jax
import jax.numpy as jnp
from jax import lax
from jax.experimental import pallas as pl
from jax.experimental.pallas import tpu as pltpu
from jax.experimental.pallas import tpu_sc as plsc

H = 256
F = 256
K = 3
NBINS = 256
SCALE = 10
B = 8
T = 2048
MAXLEN = 8192

_ZROW = B * T
_TBL_ROWS = (B + 1) * T


def _shifts(v):
    z = jnp.zeros((1, v.shape[1]), v.dtype)
    vm1 = jnp.concatenate([z, v[:-1]], axis=0)
    vp1 = jnp.concatenate([v[1:], z], axis=0)
    return vm1, vp1


def _ln(h, g, b):
    m = jnp.mean(h, axis=-1, keepdims=True)
    d = h - m
    v = jnp.mean(d * d, axis=-1, keepdims=True)
    return d * lax.rsqrt(v + 1e-5) * g + b


def _conv(x, xm1, xp1, w, b):
    y = jnp.dot(xm1, w[0], preferred_element_type=jnp.float32)
    y = y + jnp.dot(x, w[1], preferred_element_type=jnp.float32)
    y = y + jnp.dot(xp1, w[2], preferred_element_type=jnp.float32)
    return y + b


def _stack(x, xm1, xp1, c1, c2, vec, r0):
    h = jnp.maximum(_conv(x, xm1, xp1, c1, vec[r0:r0 + 1]), 0.0)
    h = _ln(h, vec[r0 + 1:r0 + 2], vec[r0 + 2:r0 + 3])
    hm1, hp1 = _shifts(h)
    h = jnp.maximum(_conv(h, hm1, hp1, c2, vec[r0 + 3:r0 + 4]), 0.0)
    h = _ln(h, vec[r0 + 4:r0 + 5], vec[r0 + 5:r0 + 6])
    return h


def _bin_onehot(tgt_col, bins_row):
    cnt = jnp.sum((bins_row < tgt_col).astype(jnp.float32), axis=-1,
                  keepdims=True)
    io = lax.broadcasted_iota(jnp.int32, (T, NBINS), 1).astype(jnp.float32)
    return (io == cnt).astype(jnp.float32)


_WREFS = 7


def _tc_body(x_ref, mask_ref, pt_ref, et_ref, dur_ref, scal_ref,
             pbins_ref, ebins_ref, pemb_ref, eemb_ref, lw_ref, vec_ref,
             wd1_ref, wd2_ref, wp1_ref, wp2_ref, wps_ref, we1_ref, we2_ref,
             x3_ref, logd_ref, pitch_ref, energy_ref, gidx_ref, misc_ref,
             ws_ref):
    bid = pl.program_id(0)

    @pl.when(bid == 0)
    def _prologue():
        io_i = lax.broadcasted_iota(jnp.int32, (H, 3 * H), 0)
        io_r = lax.broadcasted_iota(jnp.int32, (H, 3 * H), 1)
        wrefs = (wd1_ref, wd2_ref, wp1_ref, wp2_ref, wps_ref, we1_ref,
                 we2_ref)
        for k in range(K):
            sel = (io_r == 3 * io_i + k).astype(jnp.float32)
            for i, wref in enumerate(wrefs):
                ws_ref[i, k] = lax.dot_general(
                    sel, wref[...], (((1,), (1,)), ((), ())),
                    preferred_element_type=jnp.float32)
        x3_ref[0] = jnp.zeros((T, H), jnp.float32)

    @pl.when(bid > 0)
    def _main():
        x = x_ref[0]
        inv_mask = 1.0 - mask_ref[0]
        scal = scal_ref[...]
        lw = lw_ref[...]
        vec = vec_ref[...]
        xm1, xp1 = _shifts(x)

        hd = _stack(x, xm1, xp1, ws_ref[0], ws_ref[1], vec, 0)
        lind = jnp.dot(hd, lw, preferred_element_type=jnp.float32) + scal
        logd = lind[:, 0:1] * inv_mask

        hp = _stack(x, xm1, xp1, ws_ref[2], ws_ref[3], vec, 6)
        linp = jnp.dot(hp, lw, preferred_element_type=jnp.float32) + scal
        rec = linp[:, 2:3] * inv_mask
        sh = _conv(x, xm1, xp1, ws_ref[4], vec[18:19])
        sh_mean = jnp.mean(sh, axis=0, keepdims=True)
        stats = (jnp.dot(sh_mean, lw, preferred_element_type=jnp.float32)
                 + scal)
        pitch = rec + stats[:, 3:4] * stats[:, 4:5]

        ohp = _bin_onehot(pt_ref[0], pbins_ref[...])
        x2 = x + jnp.dot(ohp, pemb_ref[...],
                         preferred_element_type=jnp.float32)

        x2m1, x2p1 = _shifts(x2)
        he = _stack(x2, x2m1, x2p1, ws_ref[5], ws_ref[6], vec, 12)
        line = jnp.dot(he, lw, preferred_element_type=jnp.float32) + scal
        energy = line[:, 1:2] * inv_mask

        ohe = _bin_onehot(et_ref[0], ebins_ref[...])
        x3 = x2 + jnp.dot(ohe, eemb_ref[...],
                          preferred_element_type=jnp.float32)

        cum = dur_ref[0]
        k = 1
        while k < T:
            cum = cum + jnp.concatenate(
                [jnp.zeros((k, 1), jnp.float32), cum[:T - k]], axis=0)
            k *= 2
        mel = cum[T - 1:T, 0:1]
        cumv = cum - scal[:, 5:6]

        a = jnp.floor(cumv * (1.0 / 64.0))
        bfrac = cumv - 64.0 * a
        oh_hi = (lax.broadcasted_iota(jnp.int32, (T, 128), 1)
                 .astype(jnp.float32) == a)
        oh_lo = (lax.broadcasted_iota(jnp.int32, (T, 64), 1)
                 .astype(jnp.float32) == bfrac)
        hist2d = lax.dot_general(oh_hi.astype(jnp.float32),
                                 oh_lo.astype(jnp.float32),
                                 (((0,), (0,)), ((), ())),
                                 preferred_element_type=jnp.float32)
        u64 = (lax.broadcasted_iota(jnp.int32, (64, 64), 0)
               <= lax.broadcasted_iota(jnp.int32, (64, 64), 1))
        rowcum = jnp.dot(hist2d, u64.astype(jnp.float32),
                         preferred_element_type=jnp.float32)
        rtot2 = jnp.sum(hist2d, axis=1, keepdims=True)
        ls = (lax.broadcasted_iota(jnp.int32, (128, 128), 1)
              < lax.broadcasted_iota(jnp.int32, (128, 128), 0))
        excl = jnp.dot(ls.astype(jnp.float32), rtot2,
                       preferred_element_type=jnp.float32)
        negcnt = jnp.sum((cumv < 0.0).astype(jnp.float32), axis=0,
                         keepdims=True)
        idx2d = rowcum + excl + negcnt
        idxc = jnp.minimum(idx2d, float(T - 1)).astype(jnp.int32)

        j2d = (64 * lax.broadcasted_iota(jnp.int32, (128, 64), 0)
               + lax.broadcasted_iota(jnp.int32, (128, 64), 1)
               ).astype(jnp.float32)
        valid = j2d < (mel - scal[:, 5:6])
        zrow = _ZROW + lax.broadcasted_iota(jnp.int32, (128, 64), 1)
        gidx = jnp.where(valid, idxc + (bid - 1) * T, zrow)

        x3_ref[0] = x3
        logd_ref[0] = logd
        pitch_ref[0] = pitch
        energy_ref[0] = energy
        gidx_ref[0] = gidx
        misc_ref[0] = jnp.broadcast_to(mel, (1, 128))


def _tc_specs():
    bs = pl.BlockSpec
    bcast3 = lambda b: (0, 0, 0)
    bcast2 = lambda b: (0, 0)
    per_b3 = lambda b: (jnp.maximum(b - 1, 0), 0, 0)
    in_specs = [
        bs((1, T, H), per_b3),
        bs((1, T, 1), per_b3),
        bs((1, T, 1), per_b3),
        bs((1, T, 1), per_b3),
        bs((1, T, 1), per_b3),
        bs((1, 128), bcast2),
        bs((1, NBINS), bcast2),
        bs((1, NBINS), bcast2),
        bs((NBINS, H), bcast2),
        bs((NBINS, H), bcast2),
        bs((H, 128), bcast2),
        bs((24, H), bcast2),
    ] + [bs((H, 3 * H), bcast2)] * _WREFS
    out_specs = [
        bs((1, T, H), lambda b: (jnp.where(b == 0, B, b - 1), 0, 0)),
        bs((1, T, 1), per_b3),
        bs((1, T, 1), per_b3),
        bs((1, T, 1), per_b3),
        bs((1, 128, 64), per_b3),
        bs((1, 1, 128), per_b3),
    ]
    out_shapes = [
        jax.ShapeDtypeStruct((B + 1, T, H), jnp.float32),
        jax.ShapeDtypeStruct((B, T, 1), jnp.float32),
        jax.ShapeDtypeStruct((B, T, 1), jnp.float32),
        jax.ShapeDtypeStruct((B, T, 1), jnp.float32),
        jax.ShapeDtypeStruct((B, 128, 64), jnp.int32),
        jax.ShapeDtypeStruct((B, 1, 128), jnp.float32),
    ]
    return (B + 1,), in_specs, out_shapes, out_specs


def _prep(x, src_mask, pitch_target, energy_target, duration_target,
          max_len, params, pitch_bins, energy_bins):
    f32 = jnp.float32
    col = lambda v: v.astype(f32)[:, :, None]
    isq = 1.0 / jnp.sqrt(jnp.arange(1, SCALE + 1, dtype=f32))

    pp, dp, ep = params['pitch'], params['dur'], params['energy']
    pvec = jnp.einsum('sf,s->f', pp['lin_w'][:SCALE], isq)
    pb = jnp.dot(pp['lin_b'][:SCALE], isq)
    lw = jnp.zeros((H, 128), f32)
    lw = lw.at[:, 0].set(dp['lin_w'][0])
    lw = lw.at[:, 1].set(ep['lin_w'][0])
    lw = lw.at[:, 2].set(pvec)
    lw = lw.at[:, 3].set(pp['slin_w'][0])
    lw = lw.at[:, 4].set(pp['slin_w'][1])

    offf = jnp.asarray(max_len, f32) - MAXLEN
    scal = jnp.zeros((128,), f32)
    scal = scal.at[0].set(dp['lin_b'][0])
    scal = scal.at[1].set(ep['lin_b'][0])
    scal = scal.at[2].set(pb)
    scal = scal.at[3].set(pp['slin_b'][0])
    scal = scal.at[4].set(pp['slin_b'][1])
    scal = scal.at[5].set(offf)
    scal = scal[None, :]

    rows = [dp['conv1_b'], dp['ln1_g'], dp['ln1_b'],
            dp['conv2_b'], dp['ln2_g'], dp['ln2_b'],
            pp['conv1_b'], pp['ln1_g'], pp['ln1_b'],
            pp['conv2_b'], pp['ln2_g'], pp['ln2_b'],
            ep['conv1_b'], ep['ln1_g'], ep['ln1_b'],
            ep['conv2_b'], ep['ln2_g'], ep['ln2_b'],
            pp['sconv_b']]
    vec = jnp.stack(rows + [jnp.zeros((H,), f32)] * (24 - len(rows)))

    rw = lambda w: w.astype(f32).reshape(H, 3 * H)
    pad_bins = lambda bb: jnp.concatenate(
        [bb.astype(f32), jnp.full((1,), jnp.inf, f32)])[None, :]

    return [
        x.astype(f32),
        col(src_mask),
        col(pitch_target),
        col(energy_target),
        col(duration_target),
        scal,
        pad_bins(pitch_bins),
        pad_bins(energy_bins),
        params['pitch_emb'].astype(f32),
        params['energy_emb'].astype(f32),
        lw,
        vec,
        rw(dp['conv1_w']), rw(dp['conv2_w']),
        rw(pp['conv1_w']), rw(pp['conv2_w']), rw(pp['sconv_w']),
        rw(ep['conv1_w']), rw(ep['conv2_w']),
    ]


_SC_NC = 2
_SC_NS = 16
_NW = _SC_NC * _SC_NS
_RPW = (B * MAXLEN) // _NW
_CH = 128
_NCHUNK = _RPW // _CH
_NBUF = 3


def _sc_gather(table, gidx):
    mesh = plsc.VectorSubcoreMesh(core_axis_name="c", subcore_axis_name="s")

    @functools.partial(
        pl.kernel, mesh=mesh,
        out_type=jax.ShapeDtypeStruct((B * MAXLEN, H), jnp.float32),
        scratch_types=[pltpu.VMEM((_NCHUNK, _CH), jnp.int32)]
        + [pltpu.VMEM((_CH, H), jnp.float32)] * _NBUF
        + [pltpu.SemaphoreType.DMA] * (2 * _NBUF),
    )
    def k(table_hbm, idx_hbm, out_hbm, idx_v, *bufs_and_sems):
        rows = bufs_and_sems[:_NBUF]
        gsem = bufs_and_sems[_NBUF:2 * _NBUF]
        ssem = bufs_and_sems[2 * _NBUF:]
        wid = lax.axis_index("s") * _SC_NC + lax.axis_index("c")
        base = wid * _RPW
        pltpu.sync_copy(idx_hbm.at[wid], idx_v)
        g_cp = [None] * _NCHUNK
        s_cp = [None] * _NCHUNK
        for c in range(_NCHUNK):
            b = c % _NBUF
            if c >= _NBUF:
                s_cp[c - _NBUF].wait()
            g_cp[c] = pltpu.async_copy(table_hbm.at[idx_v.at[c]], rows[b],
                                       gsem[b])
            if c >= 1:
                bp = (c - 1) % _NBUF
                g_cp[c - 1].wait()
                s_cp[c - 1] = pltpu.async_copy(
                    rows[bp], out_hbm.at[pl.ds(base + (c - 1) * _CH, _CH)],
                    ssem[bp])
        c = _NCHUNK - 1
        g_cp[c].wait()
        s_cp[c] = pltpu.async_copy(
            rows[c % _NBUF], out_hbm.at[pl.ds(base + c * _CH, _CH)],
            ssem[c % _NBUF])
        for c in range(_NCHUNK - _NBUF, _NCHUNK):
            s_cp[c].wait()

    return k(table, gidx)


def kernel(x, src_mask, pitch_target, energy_target, duration_target,
           max_len, params, pitch_bins, energy_bins):
    grid, in_specs, out_shapes, out_specs = _tc_specs()
    args = _prep(x, src_mask, pitch_target, energy_target, duration_target,
                 max_len, params, pitch_bins, energy_bins)
    x3, logd, pitch, energy, gidx, misc = pl.pallas_call(
        _tc_body,
        grid=grid,
        in_specs=in_specs,
        out_specs=out_specs,
        out_shape=out_shapes,
        scratch_shapes=[pltpu.VMEM((_WREFS, K, H, H), jnp.float32)],
        compiler_params=pltpu.CompilerParams(
            dimension_semantics=("arbitrary",)),
    )(*args)

    table = x3.reshape(_TBL_ROWS, H)
    gidx_3d = gidx.reshape(_NW, _NCHUNK, _CH)
    out = _sc_gather(table, gidx_3d).reshape(B, MAXLEN, H)

    mel_len = misc[:, 0, 0].astype(jnp.int32)
    return (out, pitch[:, :, 0], energy[:, :, 0], logd[:, :, 0], mel_len)


# speedup vs baseline: 1.2503x; 1.1479x over previous
import functools

import jax
import jax.numpy as jnp
from jax import lax
from jax.experimental import pallas as pl
from jax.experimental.pallas import tpu as pltpu
from jax.experimental.pallas import tpu_sc as plsc

H = 256
F = 256
K = 3
NBINS = 256
SCALE = 10
B = 8
T = 2048
MAXLEN = 8192

_ZROW = B * T
_TBL_ROWS = (B + 1) * T


def _shifts(v):
    z = jnp.zeros((1, v.shape[1]), v.dtype)
    vm1 = jnp.concatenate([z, v[:-1]], axis=0)
    vp1 = jnp.concatenate([v[1:], z], axis=0)
    return vm1, vp1


def _ln(h, g, b):
    m = jnp.mean(h, axis=-1, keepdims=True)
    d = h - m
    v = jnp.mean(d * d, axis=-1, keepdims=True)
    return d * lax.rsqrt(v + 1e-5) * g + b


def _conv(x, xm1, xp1, w, b):
    bf = jnp.bfloat16
    y = jnp.dot(xm1.astype(bf), w[0], preferred_element_type=jnp.float32)
    y = y + jnp.dot(x.astype(bf), w[1], preferred_element_type=jnp.float32)
    y = y + jnp.dot(xp1.astype(bf), w[2], preferred_element_type=jnp.float32)
    return y + b


def _stack(x, xm1, xp1, c1, c2, vec, r0):
    h = jnp.maximum(_conv(x, xm1, xp1, c1, vec[r0:r0 + 1]), 0.0)
    h = _ln(h, vec[r0 + 1:r0 + 2], vec[r0 + 2:r0 + 3])
    hm1, hp1 = _shifts(h)
    h = jnp.maximum(_conv(h, hm1, hp1, c2, vec[r0 + 3:r0 + 4]), 0.0)
    h = _ln(h, vec[r0 + 4:r0 + 5], vec[r0 + 5:r0 + 6])
    return h


def _bin_onehot(tgt_col, bins_row):
    cnt = jnp.sum((bins_row < tgt_col).astype(jnp.float32), axis=-1,
                  keepdims=True)
    io = lax.broadcasted_iota(jnp.int32, (T, NBINS), 1).astype(jnp.float32)
    return (io == cnt).astype(jnp.float32)


_WREFS = 7


def _tc_body(x_ref, mask_ref, pt_ref, et_ref, dur_ref, scal_ref,
             pbins_ref, ebins_ref, pemb_ref, eemb_ref, lw_ref, vec_ref,
             wd1_ref, wd2_ref, wp1_ref, wp2_ref, wps_ref, we1_ref, we2_ref,
             x3_ref, logd_ref, pitch_ref, energy_ref, gidx_ref, misc_ref,
             ws_ref):
    bid = pl.program_id(0)

    @pl.when(bid == 0)
    def _prologue():
        io_i = lax.broadcasted_iota(jnp.int32, (H, 3 * H), 0)
        io_r = lax.broadcasted_iota(jnp.int32, (H, 3 * H), 1)
        wrefs = (wd1_ref, wd2_ref, wp1_ref, wp2_ref, wps_ref, we1_ref,
                 we2_ref)
        for k in range(K):
            sel = (io_r == 3 * io_i + k).astype(jnp.float32)
            for i, wref in enumerate(wrefs):
                ws_ref[i, k] = lax.dot_general(
                    sel, wref[...], (((1,), (1,)), ((), ())),
                    preferred_element_type=jnp.float32).astype(jnp.bfloat16)
        x3_ref[0] = jnp.zeros((T, H), jnp.float32)

    @pl.when(bid > 0)
    def _main():
        x = x_ref[0]
        inv_mask = 1.0 - mask_ref[0]
        scal = scal_ref[...]
        lw = lw_ref[...]
        vec = vec_ref[...]
        xm1, xp1 = _shifts(x)

        hd = _stack(x, xm1, xp1, ws_ref[0], ws_ref[1], vec, 0)
        lind = jnp.dot(hd, lw, preferred_element_type=jnp.float32) + scal
        logd = lind[:, 0:1] * inv_mask

        hp = _stack(x, xm1, xp1, ws_ref[2], ws_ref[3], vec, 6)
        linp = jnp.dot(hp, lw, preferred_element_type=jnp.float32) + scal
        rec = linp[:, 2:3] * inv_mask
        xbar = jnp.mean(x, axis=0, keepdims=True)
        ws4 = ws_ref[4]
        sh_mean = (jnp.dot(xbar - x[T - 1:T] * (1.0 / T), ws4[0],
                           preferred_element_type=jnp.float32)
                   + jnp.dot(xbar, ws4[1],
                             preferred_element_type=jnp.float32)
                   + jnp.dot(xbar - x[0:1] * (1.0 / T), ws4[2],
                             preferred_element_type=jnp.float32)
                   + vec[18:19])
        stats = (jnp.dot(sh_mean, lw, preferred_element_type=jnp.float32)
                 + scal)
        pitch = rec + stats[:, 3:4] * stats[:, 4:5]

        ohp = _bin_onehot(pt_ref[0], pbins_ref[...])
        x2 = x + jnp.dot(ohp, pemb_ref[...],
                         preferred_element_type=jnp.float32)

        x2m1, x2p1 = _shifts(x2)
        he = _stack(x2, x2m1, x2p1, ws_ref[5], ws_ref[6], vec, 12)
        line = jnp.dot(he, lw, preferred_element_type=jnp.float32) + scal
        energy = line[:, 1:2] * inv_mask

        ohe = _bin_onehot(et_ref[0], ebins_ref[...])
        x3 = x2 + jnp.dot(ohe, eemb_ref[...],
                          preferred_element_type=jnp.float32)

        cum = dur_ref[0]
        k = 1
        while k < T:
            cum = cum + jnp.concatenate(
                [jnp.zeros((k, 1), jnp.float32), cum[:T - k]], axis=0)
            k *= 2
        mel = cum[T - 1:T, 0:1]
        cumv = cum - scal[:, 5:6]

        a = jnp.floor(cumv * (1.0 / 64.0))
        bfrac = cumv - 64.0 * a
        oh_hi = (lax.broadcasted_iota(jnp.int32, (T, 128), 1)
                 .astype(jnp.float32) == a)
        oh_lo = (lax.broadcasted_iota(jnp.int32, (T, 64), 1)
                 .astype(jnp.float32) == bfrac)
        hist2d = lax.dot_general(oh_hi.astype(jnp.float32),
                                 oh_lo.astype(jnp.float32),
                                 (((0,), (0,)), ((), ())),
                                 preferred_element_type=jnp.float32)
        u64 = (lax.broadcasted_iota(jnp.int32, (64, 64), 0)
               <= lax.broadcasted_iota(jnp.int32, (64, 64), 1))
        rowcum = jnp.dot(hist2d, u64.astype(jnp.float32),
                         preferred_element_type=jnp.float32)
        rtot2 = jnp.sum(hist2d, axis=1, keepdims=True)
        ls = (lax.broadcasted_iota(jnp.int32, (128, 128), 1)
              < lax.broadcasted_iota(jnp.int32, (128, 128), 0))
        excl = jnp.dot(ls.astype(jnp.float32), rtot2,
                       preferred_element_type=jnp.float32)
        negcnt = jnp.sum((cumv < 0.0).astype(jnp.float32), axis=0,
                         keepdims=True)
        idx2d = rowcum + excl + negcnt
        idxc = jnp.minimum(idx2d, float(T - 1)).astype(jnp.int32)

        j2d = (64 * lax.broadcasted_iota(jnp.int32, (128, 64), 0)
               + lax.broadcasted_iota(jnp.int32, (128, 64), 1)
               ).astype(jnp.float32)
        valid = j2d < (mel - scal[:, 5:6])
        zrow = _ZROW + lax.broadcasted_iota(jnp.int32, (128, 64), 1)
        gidx = jnp.where(valid, idxc + (bid - 1) * T, zrow)

        x3_ref[0] = x3
        logd_ref[0] = logd
        pitch_ref[0] = pitch
        energy_ref[0] = energy
        gidx_ref[0] = gidx
        misc_ref[0] = jnp.broadcast_to(mel, (1, 128))


def _tc_specs():
    bs = pl.BlockSpec
    bcast3 = lambda b: (0, 0, 0)
    bcast2 = lambda b: (0, 0)
    per_b3 = lambda b: (jnp.maximum(b - 1, 0), 0, 0)
    in_specs = [
        bs((1, T, H), per_b3),
        bs((1, T, 1), per_b3),
        bs((1, T, 1), per_b3),
        bs((1, T, 1), per_b3),
        bs((1, T, 1), per_b3),
        bs((1, 128), bcast2),
        bs((1, NBINS), bcast2),
        bs((1, NBINS), bcast2),
        bs((NBINS, H), bcast2),
        bs((NBINS, H), bcast2),
        bs((H, 128), bcast2),
        bs((24, H), bcast2),
    ] + [bs((H, 3 * H), bcast2)] * _WREFS
    out_specs = [
        bs((1, T, H), lambda b: (jnp.where(b == 0, B, b - 1), 0, 0)),
        bs((1, T, 1), per_b3),
        bs((1, T, 1), per_b3),
        bs((1, T, 1), per_b3),
        bs((1, 128, 64), per_b3),
        bs((1, 1, 128), per_b3),
    ]
    out_shapes = [
        jax.ShapeDtypeStruct((B + 1, T, H), jnp.float32),
        jax.ShapeDtypeStruct((B, T, 1), jnp.float32),
        jax.ShapeDtypeStruct((B, T, 1), jnp.float32),
        jax.ShapeDtypeStruct((B, T, 1), jnp.float32),
        jax.ShapeDtypeStruct((B, 128, 64), jnp.int32),
        jax.ShapeDtypeStruct((B, 1, 128), jnp.float32),
    ]
    return (B + 1,), in_specs, out_shapes, out_specs


def _prep(x, src_mask, pitch_target, energy_target, duration_target,
          max_len, params, pitch_bins, energy_bins):
    f32 = jnp.float32
    col = lambda v: v.astype(f32)[:, :, None]
    isq = 1.0 / jnp.sqrt(jnp.arange(1, SCALE + 1, dtype=f32))

    pp, dp, ep = params['pitch'], params['dur'], params['energy']
    pvec = jnp.einsum('sf,s->f', pp['lin_w'][:SCALE], isq)
    pb = jnp.dot(pp['lin_b'][:SCALE], isq)
    lw = jnp.zeros((H, 128), f32)
    lw = lw.at[:, 0].set(dp['lin_w'][0])
    lw = lw.at[:, 1].set(ep['lin_w'][0])
    lw = lw.at[:, 2].set(pvec)
    lw = lw.at[:, 3].set(pp['slin_w'][0])
    lw = lw.at[:, 4].set(pp['slin_w'][1])

    offf = jnp.asarray(max_len, f32) - MAXLEN
    scal = jnp.zeros((128,), f32)
    scal = scal.at[0].set(dp['lin_b'][0])
    scal = scal.at[1].set(ep['lin_b'][0])
    scal = scal.at[2].set(pb)
    scal = scal.at[3].set(pp['slin_b'][0])
    scal = scal.at[4].set(pp['slin_b'][1])
    scal = scal.at[5].set(offf)
    scal = scal[None, :]

    rows = [dp['conv1_b'], dp['ln1_g'], dp['ln1_b'],
            dp['conv2_b'], dp['ln2_g'], dp['ln2_b'],
            pp['conv1_b'], pp['ln1_g'], pp['ln1_b'],
            pp['conv2_b'], pp['ln2_g'], pp['ln2_b'],
            ep['conv1_b'], ep['ln1_g'], ep['ln1_b'],
            ep['conv2_b'], ep['ln2_g'], ep['ln2_b'],
            pp['sconv_b']]
    vec = jnp.stack(rows + [jnp.zeros((H,), f32)] * (24 - len(rows)))

    rw = lambda w: w.astype(f32).reshape(H, 3 * H)
    pad_bins = lambda bb: jnp.concatenate(
        [bb.astype(f32), jnp.full((1,), jnp.inf, f32)])[None, :]

    return [
        x.astype(f32),
        col(src_mask),
        col(pitch_target),
        col(energy_target),
        col(duration_target),
        scal,
        pad_bins(pitch_bins),
        pad_bins(energy_bins),
        params['pitch_emb'].astype(f32),
        params['energy_emb'].astype(f32),
        lw,
        vec,
        rw(dp['conv1_w']), rw(dp['conv2_w']),
        rw(pp['conv1_w']), rw(pp['conv2_w']), rw(pp['sconv_w']),
        rw(ep['conv1_w']), rw(ep['conv2_w']),
    ]


_SC_NC = 2
_SC_NS = 16
_NW = _SC_NC * _SC_NS
_RPW = (B * MAXLEN) // _NW
_CH = 128
_NCHUNK = _RPW // _CH
_NBUF = 3


def _sc_gather(table, gidx):
    mesh = plsc.VectorSubcoreMesh(core_axis_name="c", subcore_axis_name="s")

    @functools.partial(
        pl.kernel, mesh=mesh,
        out_type=jax.ShapeDtypeStruct((B * MAXLEN, H), jnp.float32),
        scratch_types=[pltpu.VMEM((_NCHUNK, _CH), jnp.int32)]
        + [pltpu.VMEM((_CH, H), jnp.float32)] * _NBUF
        + [pltpu.SemaphoreType.DMA] * (2 * _NBUF),
    )
    def k(table_hbm, idx_hbm, out_hbm, idx_v, *bufs_and_sems):
        rows = bufs_and_sems[:_NBUF]
        gsem = bufs_and_sems[_NBUF:2 * _NBUF]
        ssem = bufs_and_sems[2 * _NBUF:]
        wid = lax.axis_index("s") * _SC_NC + lax.axis_index("c")
        base = wid * _RPW
        pltpu.sync_copy(idx_hbm.at[wid], idx_v)
        g_cp = [None] * _NCHUNK
        s_cp = [None] * _NCHUNK
        for c in range(_NCHUNK):
            b = c % _NBUF
            if c >= _NBUF:
                s_cp[c - _NBUF].wait()
            g_cp[c] = pltpu.async_copy(table_hbm.at[idx_v.at[c]], rows[b],
                                       gsem[b])
            if c >= 1:
                bp = (c - 1) % _NBUF
                g_cp[c - 1].wait()
                s_cp[c - 1] = pltpu.async_copy(
                    rows[bp], out_hbm.at[pl.ds(base + (c - 1) * _CH, _CH)],
                    ssem[bp])
        c = _NCHUNK - 1
        g_cp[c].wait()
        s_cp[c] = pltpu.async_copy(
            rows[c % _NBUF], out_hbm.at[pl.ds(base + c * _CH, _CH)],
            ssem[c % _NBUF])
        for c in range(_NCHUNK - _NBUF, _NCHUNK):
            s_cp[c].wait()

    return k(table, gidx)


def kernel(x, src_mask, pitch_target, energy_target, duration_target,
           max_len, params, pitch_bins, energy_bins):
    grid, in_specs, out_shapes, out_specs = _tc_specs()
    args = _prep(x, src_mask, pitch_target, energy_target, duration_target,
                 max_len, params, pitch_bins, energy_bins)
    x3, logd, pitch, energy, gidx, misc = pl.pallas_call(
        _tc_body,
        grid=grid,
        in_specs=in_specs,
        out_specs=out_specs,
        out_shape=out_shapes,
        scratch_shapes=[pltpu.VMEM((_WREFS, K, H, H), jnp.bfloat16)],
        compiler_params=pltpu.CompilerParams(
            dimension_semantics=("arbitrary",)),
    )(*args)

    table = x3.reshape(_TBL_ROWS, H)
    gidx_3d = gidx.reshape(_NW, _NCHUNK, _CH)
    out = _sc_gather(table, gidx_3d).reshape(B, MAXLEN, H)

    mel_len = misc[:, 0, 0].astype(jnp.int32)
    return (out, pitch[:, :, 0], energy[:, :, 0], logd[:, :, 0], mel_len)
